# baseline (speedup 1.0000x reference)
import jax
import jax.numpy as jnp
import numpy as np
from jax.experimental import pallas as pl

_LOG2 = float(np.log(2.0))


def _mlp_pool_kernel(x_ref, mask_ref, w1_ref, b1_ref, w2_ref, b2_ref, out_ref):
    bb, a, d = x_ref.shape
    n = bb * a
    x = x_ref[...].reshape(n, d)
    h = jnp.dot(x, w1_ref[...], preferred_element_type=jnp.float32) + b1_ref[...]
    sp = (jnp.maximum(h, 0.0) + jnp.log1p(jnp.exp(-jnp.abs(h)))) - _LOG2
    mask = mask_ref[...]
    mask_tiled = jnp.concatenate([mask] * bb, axis=1)
    seg = jax.lax.broadcasted_iota(jnp.int32, (bb, n), 1) // a
    row = jax.lax.broadcasted_iota(jnp.int32, (bb, n), 0)
    mt = jnp.where(seg == row, mask_tiled, 0.0)
    q = jnp.dot(mt, sp, preferred_element_type=jnp.float32)
    y = jnp.sum(q * w2_ref[...], axis=1, keepdims=True)
    msum = jnp.sum(mask, axis=1, keepdims=True)
    out_ref[...] = y + b2_ref[0, 0] * msum


def kernel(representation, atomic_numbers, atom_mask, W1, b1, W2, b2):
    B, A, D = representation.shape
    H = W1.shape[1]
    BB = 16

    y = pl.pallas_call(
        _mlp_pool_kernel,
        grid=(B // BB,),
        in_specs=[
            pl.BlockSpec((BB, A, D), lambda i: (i, 0, 0)),
            pl.BlockSpec((BB, A), lambda i: (i, 0)),
            pl.BlockSpec((D, H), lambda i: (0, 0)),
            pl.BlockSpec((1, H), lambda i: (0, 0)),
            pl.BlockSpec((1, H), lambda i: (0, 0)),
            pl.BlockSpec((1, 1), lambda i: (0, 0)),
        ],
        out_specs=pl.BlockSpec((BB, 1), lambda i: (i, 0)),
        out_shape=jax.ShapeDtypeStruct((B, 1), jnp.float32),
    )(representation, atom_mask, W1, b1.reshape(1, H), W2.reshape(1, H),
      b2.reshape(1, 1))
    return y


# speedup vs baseline: 1.2753x; 1.2753x over previous
import jax
import jax.numpy as jnp
import numpy as np
from jax.experimental import pallas as pl

_LOG2 = float(np.log(2.0))


def _mlp_pool_kernel(x_ref, mask_ref, w1_ref, b1_ref, w2_ref, b2_ref, out_ref):
    bb, a, d = x_ref.shape
    n = bb * a
    hb = bb // 2
    m = n // 2
    x = x_ref[...].reshape(n, d)
    w1 = w1_ref[...]
    b1 = b1_ref[...]
    h1 = jnp.dot(x[:m, :], w1, preferred_element_type=jnp.float32) + b1
    h2 = jnp.dot(x[m:, :], w1, preferred_element_type=jnp.float32) + b1
    hp = jnp.concatenate([h1, h2], axis=1)
    sp = (jnp.maximum(hp, 0.0) + jnp.log1p(jnp.exp(-jnp.abs(hp)))) - _LOG2
    mask = mask_ref[...]
    seg = jax.lax.broadcasted_iota(jnp.int32, (hb, m), 1) // a
    row = jax.lax.broadcasted_iota(jnp.int32, (hb, m), 0)
    hit = seg == row
    mta = jnp.where(hit, jnp.concatenate([mask[:hb]] * hb, axis=1), 0.0)
    mtb = jnp.where(hit, jnp.concatenate([mask[hb:]] * hb, axis=1), 0.0)
    qa = jnp.dot(mta, sp, preferred_element_type=jnp.float32)
    qb = jnp.dot(mtb, sp, preferred_element_type=jnp.float32)
    nh = h1.shape[1]
    w2 = w2_ref[...]
    ya = jnp.sum(qa[:, :nh] * w2, axis=1, keepdims=True)
    yb = jnp.sum(qb[:, nh:] * w2, axis=1, keepdims=True)
    y = jnp.concatenate([ya, yb], axis=0)
    msum = jnp.sum(mask, axis=1, keepdims=True)
    out_ref[...] = y + b2_ref[0, 0] * msum


def kernel(representation, atomic_numbers, atom_mask, W1, b1, W2, b2):
    B, A, D = representation.shape
    H = W1.shape[1]
    BB = 16

    y = pl.pallas_call(
        _mlp_pool_kernel,
        grid=(B // BB,),
        in_specs=[
            pl.BlockSpec((BB, A, D), lambda i: (i, 0, 0)),
            pl.BlockSpec((BB, A), lambda i: (i, 0)),
            pl.BlockSpec((D, H), lambda i: (0, 0)),
            pl.BlockSpec((1, H), lambda i: (0, 0)),
            pl.BlockSpec((1, H), lambda i: (0, 0)),
            pl.BlockSpec((1, 1), lambda i: (0, 0)),
        ],
        out_specs=pl.BlockSpec((BB, 1), lambda i: (i, 0)),
        out_shape=jax.ShapeDtypeStruct((B, 1), jnp.float32),
    )(representation, atom_mask, W1, b1.reshape(1, H), W2.reshape(1, H),
      b2.reshape(1, 1))
    return y
